# baseline (speedup 1.0000x reference)
import functools

import jax
import jax.numpy as jnp
from jax import lax
from jax.experimental import pallas as pl
from jax.experimental.pallas import tpu as pltpu
from jax.experimental.pallas import tpu_sc as plsc

B = 4096
D = 64
K = 5
V = 100000
NC = 2
NS = 16
L = 16
NW = NC * NS
BW = B // NW
TW = 512
TG = (V + TW - 1) // TW


def _tc_transpose(t_t):
    def body(x_ref, o_ref):
        o_ref[...] = x_ref[...].T

    return pl.pallas_call(
        body,
        grid=(TG,),
        in_specs=[pl.BlockSpec((D, TW), lambda i: (0, i))],
        out_specs=pl.BlockSpec((TW, D), lambda i: (i, 0)),
        out_shape=jax.ShapeDtypeStruct((V, D), jnp.float32),
    )(t_t)


def _sc_gather(ids, table, nchunks):
    mesh = plsc.VectorSubcoreMesh(
        core_axis_name="c", subcore_axis_name="s", num_cores=NC, num_subcores=NS
    )

    @functools.partial(
        pl.kernel,
        out_type=jax.ShapeDtypeStruct((nchunks * B, D), jnp.float32),
        mesh=mesh,
        scratch_types=[
            pltpu.VMEM((nchunks * BW,), jnp.int32),
            pltpu.VMEM((2, BW, D), jnp.float32),
            pltpu.SemaphoreType.DMA,
            pltpu.SemaphoreType.DMA,
            pltpu.SemaphoreType.DMA,
        ],
        compiler_params=pltpu.CompilerParams(use_tc_tiling_on_sc=True),
    )
    def body(ids_hbm, tbl, out_hbm, ids_v, bufs, gsem, osem0, osem1):
        osems = (osem0, osem1)
        wid = lax.axis_index("s") * NC + lax.axis_index("c")
        base = wid * BW

        for c in range(nchunks):
            pltpu.sync_copy(ids_hbm.at[pl.ds(c * B + base, BW)],
                            ids_v.at[pl.ds(c * BW, BW)])

        out_cps = [None, None]
        for c in range(nchunks):
            buf = bufs.at[c % 2]
            if out_cps[c % 2] is not None:
                out_cps[c % 2].wait()

            def fire(ci, carry, c=c, buf=buf):
                idvec = ids_v[pl.ds(c * BW + ci * L, L)]
                for j in range(L):
                    rid = idvec[j]
                    pltpu.async_copy(tbl.at[pl.ds(rid, 1), :],
                                     buf.at[pl.ds(ci * L + j, 1), :], gsem)
                return carry

            lax.fori_loop(0, BW // L, fire, 0)
            pltpu.make_async_copy(tbl.at[pl.ds(0, BW), :], buf, gsem).wait()
            out_cps[c % 2] = pltpu.async_copy(
                buf,
                out_hbm.at[pl.ds(pl.multiple_of(c * B + base, 128), BW), :],
                osems[c % 2])
        for cp in out_cps:
            if cp is not None:
                cp.wait()

    return body(ids, table)


def _tc_loss(rows_u, rows_rest):
    def tc_body(u_ref, rest_ref, o_ref):
        u = u_ref[...]
        rest = rest_ref[...]
        ut = jnp.concatenate([u] * (K + 1), axis=0)
        q = rest * ut
        ones_m = jnp.ones((D, 128), jnp.float32)
        s = jax.lax.dot_general(
            q, ones_m, (((1,), (0,)), ((), ())),
            preferred_element_type=jnp.float32)
        rid = lax.broadcasted_iota(jnp.int32, ((K + 1) * B, 128), 0)
        s = jnp.where(rid < B, s, -s)
        t = jnp.log(1.0 / (1.0 + jnp.exp(-s)) + 1e-10)
        o_ref[0, 0] = -jnp.sum(t) / jnp.float32(128 * B)

    return pl.pallas_call(
        tc_body,
        out_shape=jax.ShapeDtypeStruct((1, 1), jnp.float32),
        out_specs=pl.BlockSpec(memory_space=pltpu.SMEM),
    )(rows_u, rows_rest)


def kernel(user_ids, pos_book_ids, neg_book_ids, user_embed, book_embed):
    uid = user_ids.astype(jnp.int32)
    pid = pos_book_ids.astype(jnp.int32)
    nid_flat = neg_book_ids.astype(jnp.int32).T.reshape(K * B)
    user_rm = _tc_transpose(user_embed.T)
    rows_u = _sc_gather(uid, user_rm, 1)
    book_rm = _tc_transpose(book_embed.T)
    rows_rest = _sc_gather(jnp.concatenate([pid, nid_flat]), book_rm, K + 1)
    loss = _tc_loss(rows_u, rows_rest)
    return loss.reshape(())


# speedup vs baseline: 2.4849x; 2.4849x over previous
import functools

import jax
import jax.numpy as jnp
from jax import lax
from jax.experimental import pallas as pl
from jax.experimental.pallas import tpu as pltpu
from jax.experimental.pallas import tpu_sc as plsc

B = 4096
D = 64
K = 5
NC = 2
NS = 16
L = 16
NW = NC * NS
BW = B // NW
R = B * (K + 2)
RP = R // 2


def _sc_gather(uid, pid, nid_flat, user_embed, book_embed):
    mesh = plsc.VectorSubcoreMesh(
        core_axis_name="c", subcore_axis_name="s", num_cores=NC, num_subcores=NS
    )

    @functools.partial(
        pl.kernel,
        out_type=jax.ShapeDtypeStruct((R, D), jnp.float32),
        mesh=mesh,
        scratch_types=[
            pltpu.VMEM((BW,), jnp.int32),
            pltpu.VMEM((BW,), jnp.int32),
            pltpu.VMEM((K * BW,), jnp.int32),
            pltpu.VMEM((K + 2, BW, D), jnp.float32),
            pltpu.SemaphoreType.DMA,
            pltpu.SemaphoreType.DMA,
        ] + [pltpu.SemaphoreType.DMA] * (K + 2),
        compiler_params=pltpu.CompilerParams(use_tc_tiling_on_sc=True),
    )
    def body(uid_hbm, pid_hbm, nid_hbm, uemb_hbm, bemb_hbm, out_hbm,
             uid_v, pid_v, nid_v, bufs, isem, osem, *gsems):
        wid = lax.axis_index("s") * NC + lax.axis_index("c")
        base = wid * BW

        id_cps = [
            pltpu.async_copy(uid_hbm.at[pl.ds(base, BW)], uid_v, isem),
            pltpu.async_copy(pid_hbm.at[pl.ds(base, BW)], pid_v, isem),
        ]
        for kk in range(K):
            id_cps.append(pltpu.async_copy(
                nid_hbm.at[pl.ds(kk * B + base, BW)],
                nid_v.at[pl.ds(kk * BW, BW)], isem))
        for cp in id_cps:
            cp.wait()

        chunks = [(uid_v, 0, uemb_hbm, base), (pid_v, 0, bemb_hbm, B + base)]
        for kk in range(K):
            chunks.append((nid_v, kk * BW, bemb_hbm, (2 + kk) * B + base))

        for c, (idref, idoff, tbl, obase) in enumerate(chunks):
            buf = bufs.at[c]
            gsem = gsems[c]

            def fire(ci, carry, idref=idref, idoff=idoff, tbl=tbl, buf=buf,
                     gsem=gsem):
                idvec = idref[pl.ds(idoff + ci * L, L)]
                for j in range(L):
                    rid = idvec[j]
                    pltpu.async_copy(tbl.at[pl.ds(rid, 1), :],
                                     buf.at[pl.ds(ci * L + j, 1), :], gsem)
                return carry

            lax.fori_loop(0, BW // L, fire, 0)

        out_cps = []
        for c, (idref, idoff, tbl, obase) in enumerate(chunks):
            buf = bufs.at[c]
            pltpu.make_async_copy(
                uemb_hbm.at[pl.ds(0, BW), :], buf, gsems[c]).wait()
            out_cps.append(pltpu.async_copy(
                buf,
                out_hbm.at[pl.ds(pl.multiple_of(obase, 128), BW), :],
                osem))
        for cp in out_cps:
            cp.wait()

    return body(uid, pid, nid_flat, user_embed, book_embed)


def _tc_loss(rows):
    def tc_body(rows_ref, o_ref):
        u = rows_ref[0:B, :]
        rest = rows_ref[B:(2 + K) * B, :]
        ut = jnp.concatenate([u] * (K + 1), axis=0)
        q = rest * ut
        ones_m = jnp.ones((D, 128), jnp.float32)
        s = jax.lax.dot_general(
            q, ones_m, (((1,), (0,)), ((), ())),
            preferred_element_type=jnp.float32)
        rid = lax.broadcasted_iota(jnp.int32, ((K + 1) * B, 128), 0)
        s = jnp.where(rid < B, s, -s)
        t = jnp.log(1.0 / (1.0 + jnp.exp(-s)) + 1e-10)
        o_ref[0, 0] = -jnp.sum(t) / jnp.float32(128 * B)

    return pl.pallas_call(
        tc_body,
        out_shape=jax.ShapeDtypeStruct((1, 1), jnp.float32),
        out_specs=pl.BlockSpec(memory_space=pltpu.SMEM),
    )(rows)


def kernel(user_ids, pos_book_ids, neg_book_ids, user_embed, book_embed):
    uid = user_ids.astype(jnp.int32)
    pid = pos_book_ids.astype(jnp.int32)
    nid_flat = neg_book_ids.astype(jnp.int32).T.reshape(K * B)
    rows = _sc_gather(uid, pid, nid_flat, user_embed, book_embed)
    loss = _tc_loss(rows)
    return loss.reshape(())
